# baseline (speedup 1.0000x reference)
import functools

import jax
import jax.numpy as jnp
from jax import lax
from jax.experimental import pallas as pl
from jax.experimental.pallas import tpu as pltpu
from jax.experimental.pallas import tpu_sc as plsc

_VOCAB = 100000
_OUT = 64
_B = 1024
_L = 200

_NC = 2
_NS = 16
_NW = _NC * _NS
_RPW = _B // _NW
_LANES = 16
_CHUNKS = _OUT // _LANES
_UNROLL = 8


_NBUF = 8


def _bow_body(tok_hbm, w_hbm, b_hbm, out_hbm, idx_v, *scratch):
    rows_bufs = scratch[:_NBUF]
    b_v, out_v = scratch[_NBUF], scratch[_NBUF + 1]
    sems = scratch[_NBUF + 2:]
    wid = lax.axis_index("s") * _NC + lax.axis_index("c")
    base = wid * _RPW

    pltpu.sync_copy(tok_hbm.at[pl.ds(base, _RPW)], idx_v)
    pltpu.sync_copy(b_hbm, b_v)

    bias = [b_v[pl.ds(c * _LANES, _LANES)] for c in range(_CHUNKS)]

    _SPLIT_A = 96

    def gathers(r, buf, sa, sb):
        return (
            pltpu.make_async_copy(
                w_hbm.at[idx_v.at[r, pl.ds(0, _SPLIT_A)]],
                buf.at[pl.ds(0, _SPLIT_A)], sa),
            pltpu.make_async_copy(
                w_hbm.at[idx_v.at[r, pl.ds(_SPLIT_A, _L - _SPLIT_A)]],
                buf.at[pl.ds(_SPLIT_A, _L - _SPLIT_A)], sb),
        )

    bufs = tuple(zip(rows_bufs, sems[::2], sems[1::2]))
    for k, (buf, sa, sb) in enumerate(bufs):
        for c in gathers(k, buf, sa, sb):
            c.start()

    @pl.loop(0, _RPW, step=len(bufs))
    def _(r):
        for k, (buf, sa, sb) in enumerate(bufs):
            rr = r + k
            for c in gathers(rr, buf, sa, sb):
                c.wait()

            def acc_body(j, accs, buf=buf):
                accs = list(accs)
                row = j * _UNROLL
                for u in range(_UNROLL):
                    for c in range(_CHUNKS):
                        a = (u % 2) * _CHUNKS + c
                        accs[a] = accs[a] + buf[row + u, pl.ds(c * _LANES, _LANES)]
                return tuple(accs)

            zero = jnp.zeros((_LANES,), jnp.float32)
            accs = lax.fori_loop(0, _L // _UNROLL, acc_body, (zero,) * (2 * _CHUNKS))
            accs = [accs[c] + accs[_CHUNKS + c] for c in range(_CHUNKS)]

            @pl.when(rr + len(bufs) < _RPW)
            def _(buf=buf, sa=sa, sb=sb, rr=rr):
                for c in gathers(rr + len(bufs), buf, sa, sb):
                    c.start()

            for c in range(_CHUNKS):
                out_v[rr, pl.ds(c * _LANES, _LANES)] = jnp.maximum(
                    accs[c] + bias[c], 0.0
                )

    pltpu.sync_copy(out_v, out_hbm.at[pl.ds(base, _RPW)])


_TCV = 4096
_SPLIT = 13 * _TCV


def _linearize_body(lo_ref, hi_ref, out_ref):
    i = pl.program_id(0)
    lo = lo_ref[:, pl.ds(i * _TCV, _TCV)]
    hi = hi_ref[:, pl.ds(i * _TCV, _TCV)]
    z = jnp.concatenate([lo, hi], axis=0)
    out_ref[...] = jnp.transpose(z, (1, 0))


def _linearize(Wt):
    return pl.pallas_call(
        _linearize_body,
        out_shape=jax.ShapeDtypeStruct((_SPLIT, 2 * _OUT), jnp.float32),
        grid=(_SPLIT // _TCV,),
        in_specs=[
            pl.BlockSpec((_OUT, _SPLIT), lambda i: (0, 0)),
            pl.BlockSpec((_OUT, _SPLIT), lambda i: (0, 1)),
        ],
        out_specs=pl.BlockSpec((_TCV, 2 * _OUT), lambda i: (i, 0)),
    )(Wt, Wt)


@jax.jit
def kernel(tokens, W, b):
    tok = tokens.astype(jnp.int32)
    tok = jnp.where(tok < _SPLIT, 2 * tok, 2 * (tok - _SPLIT) + 1)
    w_lin = _linearize(W.T).reshape(2 * _SPLIT, _OUT)
    run = functools.partial(
        pl.kernel,
        out_type=jax.ShapeDtypeStruct((_B, _OUT), jnp.float32),
        mesh=plsc.VectorSubcoreMesh(core_axis_name="c", subcore_axis_name="s"),
        scratch_types=[
            pltpu.VMEM((_RPW, _L), jnp.int32),
            *[pltpu.VMEM((_L, _OUT), jnp.float32) for _ in range(_NBUF)],
            pltpu.VMEM((_OUT,), jnp.float32),
            pltpu.VMEM((_RPW, _OUT), jnp.float32),
            *[pltpu.SemaphoreType.DMA for _ in range(2 * _NBUF)],
        ],
        compiler_params=pltpu.CompilerParams(use_tc_tiling_on_sc=False),
    )(_bow_body)
    return run(tok, w_lin, b)


# speedup vs baseline: 1.0331x; 1.0331x over previous
import functools

import jax
import jax.numpy as jnp
from jax import lax
from jax.experimental import pallas as pl
from jax.experimental.pallas import tpu as pltpu
from jax.experimental.pallas import tpu_sc as plsc

_VOCAB = 100000
_OUT = 64
_B = 1024
_L = 200

_NC = 2
_NS = 16
_NW = _NC * _NS
_RPW = _B // _NW
_LANES = 16
_CHUNKS = _OUT // _LANES
_UNROLL = 8


_NBUF = 8


def _bow_body(tok_hbm, w_hbm, b_hbm, out_hbm, idx_v, *scratch):
    rows_bufs = scratch[:_NBUF]
    b_v, out_v = scratch[_NBUF], scratch[_NBUF + 1]
    sems = scratch[_NBUF + 2:]
    wid = lax.axis_index("s") * _NC + lax.axis_index("c")
    base = wid * _RPW

    pltpu.sync_copy(tok_hbm.at[pl.ds(base, _RPW)], idx_v)
    pltpu.sync_copy(b_hbm, b_v)

    bias = [b_v[pl.ds(c * _LANES, _LANES)] for c in range(_CHUNKS)]

    def gather(r, buf, sem):
        return pltpu.make_async_copy(
            w_hbm.at[idx_v.at[r]], buf, sem
        )

    bufs = tuple(zip(rows_bufs, sems))
    for k, (buf, sem) in enumerate(bufs):
        gather(k, buf, sem).start()

    @pl.loop(0, _RPW, step=len(bufs))
    def _(r):
        for k, (buf, sem) in enumerate(bufs):
            rr = r + k
            gather(rr, buf, sem).wait()

            def acc_body(j, accs, buf=buf):
                accs = list(accs)
                row = j * _UNROLL
                for u in range(_UNROLL):
                    for c in range(_CHUNKS):
                        a = (u % 2) * _CHUNKS + c
                        accs[a] = accs[a] + buf[row + u, pl.ds(c * _LANES, _LANES)]
                return tuple(accs)

            zero = jnp.zeros((_LANES,), jnp.float32)
            accs = lax.fori_loop(0, _L // _UNROLL, acc_body, (zero,) * (2 * _CHUNKS))
            accs = [accs[c] + accs[_CHUNKS + c] for c in range(_CHUNKS)]

            @pl.when(rr + len(bufs) < _RPW)
            def _(buf=buf, sem=sem, rr=rr):
                gather(rr + len(bufs), buf, sem).start()

            for c in range(_CHUNKS):
                out_v[rr, pl.ds(c * _LANES, _LANES)] = jnp.maximum(
                    accs[c] + bias[c], 0.0
                )

    pltpu.sync_copy(out_v, out_hbm.at[pl.ds(base, _RPW)])


_TCV = 8192
_SPLIT = 7 * _TCV


def _linearize_body(lo_ref, hi_ref, out_ref):
    i = pl.program_id(0)
    lo = lo_ref[:, pl.ds(i * _TCV, _TCV)]
    hi = hi_ref[:, pl.ds(i * _TCV, _TCV)]
    z = jnp.concatenate([lo, hi], axis=0)
    out_ref[...] = jnp.transpose(z, (1, 0))


def _linearize(Wt):
    return pl.pallas_call(
        _linearize_body,
        out_shape=jax.ShapeDtypeStruct((_SPLIT, 2 * _OUT), jnp.float32),
        grid=(_SPLIT // _TCV,),
        in_specs=[
            pl.BlockSpec((_OUT, _SPLIT), lambda i: (0, 0)),
            pl.BlockSpec((_OUT, _SPLIT), lambda i: (0, 1)),
        ],
        out_specs=pl.BlockSpec((_TCV, 2 * _OUT), lambda i: (i, 0)),
    )(Wt, Wt)


@jax.jit
def kernel(tokens, W, b):
    tok = tokens.astype(jnp.int32)
    tok = jnp.where(tok < _SPLIT, 2 * tok, 2 * (tok - _SPLIT) + 1)
    w_lin = _linearize(W.T).reshape(2 * _SPLIT, _OUT)
    run = functools.partial(
        pl.kernel,
        out_type=jax.ShapeDtypeStruct((_B, _OUT), jnp.float32),
        mesh=plsc.VectorSubcoreMesh(core_axis_name="c", subcore_axis_name="s"),
        scratch_types=[
            pltpu.VMEM((_RPW, _L), jnp.int32),
            *[pltpu.VMEM((_L, _OUT), jnp.float32) for _ in range(_NBUF)],
            pltpu.VMEM((_OUT,), jnp.float32),
            pltpu.VMEM((_RPW, _OUT), jnp.float32),
            *[pltpu.SemaphoreType.DMA for _ in range(_NBUF)],
        ],
        compiler_params=pltpu.CompilerParams(use_tc_tiling_on_sc=False),
    )(_bow_body)
    return run(tok, w_lin, b)
